# baseline (speedup 1.0000x reference)
import functools

import jax
import jax.numpy as jnp
from jax import lax
from jax.experimental import pallas as pl
from jax.experimental.pallas import tpu as pltpu
from jax.experimental.pallas import tpu_sc as plsc

_NC, _NS, _VEC = 2, 16, 16
_NW = _NC * _NS
_CH = 128
_SCH = 2 * _CH


def _make_sc_kernel(tok, t, c):
    per_w = tok // _NW
    nch = per_w // _CH
    nsc = per_w // _SCH
    nbt = 10
    nr = 2 * t // nbt
    mesh = plsc.VectorSubcoreMesh(
        core_axis_name="c", subcore_axis_name="s",
        num_cores=_NC, num_subcores=_NS,
    )

    @functools.partial(
        pl.kernel,
        out_type=jax.ShapeDtypeStruct((tok, c), jnp.float32),
        mesh=mesh,
        scratch_types=[
            pltpu.VMEM_SHARED((2 * t, c), jnp.float32),
            pltpu.VMEM((2 * t // 10, c), jnp.float32),
            pltpu.VMEM((2 * t // 10, c), jnp.float32),
            pltpu.VMEM((2, c), jnp.float32),
            pltpu.VMEM((per_w,), jnp.int32),
            pltpu.VMEM((nch, _CH), jnp.int32),
            pltpu.VMEM((2, _SCH, c), jnp.float32),
            pltpu.SemaphoreType.DMA,
            pltpu.SemaphoreType.DMA,
        ],
    )
    def sc_kernel(pos_hbm, tab_hbm, mask_hbm, out_hbm,
                  comb_sh, cbuf, pbuf, tbuf, mask_v, idx_v, bufs, sem0, sem1):
        sid = lax.axis_index("s")
        wid = sid * _NC + lax.axis_index("c")
        base = wid * per_w

        @pl.when(sid < nbt)
        def _():
            mrow = sid // (nbt // 2)
            tr0 = (sid - mrow * (nbt // 2)) * nr
            pltpu.sync_copy(pos_hbm.at[pl.ds(tr0, nr)], pbuf)
            pltpu.sync_copy(tab_hbm, tbuf)

            def comb_row(r, _):
                def comb_vec(v, _):
                    o = v * _VEC
                    cbuf[r, pl.ds(o, _VEC)] = (
                        pbuf[r, pl.ds(o, _VEC)] + tbuf[mrow, pl.ds(o, _VEC)])
                    return 0
                return lax.fori_loop(0, c // _VEC, comb_vec, 0)

            lax.fori_loop(0, nr, comb_row, 0)
            pltpu.sync_copy(cbuf, comb_sh.at[pl.ds(sid * nr, nr)])

        pltpu.sync_copy(mask_hbm.at[pl.ds(base, per_w)], mask_v)

        lanes = lax.iota(jnp.int32, _VEC)

        def idx_row(j, _):
            def idx_vec(v, _):
                p = j * _CH + v * _VEC
                m = mask_v[pl.ds(p, _VEC)]
                tpos = lax.rem(base + p + lanes, t)
                idx_v[j, pl.ds(v * _VEC, _VEC)] = m * t + tpos
                return 0
            return lax.fori_loop(0, _CH // _VEC, idx_vec, 0)

        lax.fori_loop(0, nch, idx_row, 0)

        plsc.subcore_barrier()

        b0 = bufs.at[0]
        b1 = bufs.at[1]

        def fire(s, buf, sem):
            pltpu.async_copy(comb_sh.at[idx_v.at[2 * s]],
                             buf.at[pl.ds(0, _CH)], sem)
            pltpu.async_copy(comb_sh.at[idx_v.at[2 * s + 1]],
                             buf.at[pl.ds(_CH, _CH)], sem)

        def drain(buf, sem):
            pltpu.make_async_copy(out_hbm.at[pl.ds(0, _SCH)], buf, sem).wait()

        def scatter(s, buf):
            pltpu.sync_copy(buf, out_hbm.at[pl.ds(base + s * _SCH, _SCH)])

        fire(0, b0, sem0)
        fire(1, b1, sem1)

        def pair(g, _):
            s0 = 2 * g
            s1 = s0 + 1
            drain(b0, sem0)
            scatter(s0, b0)

            @pl.when(s0 + 2 < nsc)
            def _():
                fire(s0 + 2, b0, sem0)

            drain(b1, sem1)
            scatter(s1, b1)

            @pl.when(s1 + 2 < nsc)
            def _():
                fire(s1 + 2, b1, sem1)
            return 0

        lax.fori_loop(0, nsc // 2, pair, 0)

        drain(b0, sem0)
        scatter(nsc - 1, b0)

    return sc_kernel


def kernel(x, mask, pos_emb, mask_table):
    b, t, c = x.shape
    tok = b * t
    pos = pos_emb[0, :t, :]
    m_flat = mask.astype(jnp.int32).reshape(tok)
    out = _make_sc_kernel(tok, t, c)(pos, mask_table, m_flat)
    return out.reshape(b, t, c)
